# baseline (speedup 1.0000x reference)
import jax
import jax.numpy as jnp
from jax import lax
from jax.experimental import pallas as pl
from jax.experimental.pallas import tpu as pltpu
from jax.experimental.pallas import tpu_sc as plsc

NU = 25000
NI = 25000
NP = 25088
DIM = 64
E = 800000
ET = 50176
EP = ET * 16
ER = EP // 128
RPT = ER // 16
NB = 2
NG = RPT // NB
CG = 14
NCH = RPT // CG
NPT = NP // 16
CH = 112
BATCH = 4096
PB = BATCH // 32
L = 16


def _rsqrt16(x):
    i = lax.bitcast_convert_type(x, jnp.int32)
    i = jnp.int32(0x5F3759DF) - lax.shift_right_arithmetic(i, 1)
    y = lax.bitcast_convert_type(i, jnp.float32)
    for _ in range(3):
        y = y * (jnp.float32(1.5) - jnp.float32(0.5) * x * y * y)
    return y


def _setup_body(src_hbm, dst_hbm, ue_hbm, ie_hbm,
                su_hbm, si_hbm, xsu_hbm, xsi_hbm,
                deg_sh, idxc, onesv, degv, svecv, rowsv, semi, sems):
    c = lax.axis_index("c")
    t = lax.axis_index("s")
    nbase = t * NPT
    rbase = t * RPT

    for k in range(128 // L):
        onesv[pl.ds(k * L, L)] = jnp.ones((L,), jnp.float32)

    def zloop(i, carry):
        svecv[pl.ds(i * L, L)] = jnp.zeros((L,), jnp.float32)
        return carry
    lax.fori_loop(0, NPT // L, zloop, None)
    pltpu.sync_copy(svecv, deg_sh.at[pl.ds(nbase, NPT)])
    plsc.subcore_barrier()

    def half(idx_hbm, emb_hbm, s_hbm, xs_hbm):
        def gloop(g, carry):
            rb = rbase + g * CG
            pltpu.async_copy(idx_hbm.at[pl.ds(rb, CG)], idxc, semi).wait()
            for j in range(CG):
                pltpu.async_copy(onesv, deg_sh.at[idxc.at[j]], sems,
                                 add=True).wait()
            return carry
        lax.fori_loop(0, RPT // CG, gloop, None)
        plsc.subcore_barrier()

        pltpu.sync_copy(deg_sh.at[pl.ds(nbase, NPT)], degv)

        def sloop(i, carry):
            d = jnp.maximum(degv[pl.ds(i * L, L)], jnp.float32(1.0))
            svecv[pl.ds(i * L, L)] = _rsqrt16(d)
            return carry
        lax.fori_loop(0, NPT // L, sloop, None)
        pltpu.sync_copy(svecv, s_hbm.at[pl.ds(nbase, NPT)])

        def chunk(cix, carry):
            r0 = nbase + cix * CH
            pltpu.sync_copy(emb_hbm.at[pl.ds(r0, CH)], rowsv)

            def rgroup(i16, carry2):
                s16 = svecv[pl.ds(cix * CH + i16 * L, L)]
                for r in range(L):
                    row = i16 * L + r
                    sv = s16[r]
                    for k in range(DIM // L):
                        sl = pl.ds(k * L, L)
                        rowsv[row, sl] = rowsv[row, sl] * sv
                return carry2
            lax.fori_loop(0, CH // L, rgroup, None)
            pltpu.sync_copy(rowsv, xs_hbm.at[pl.ds(r0, CH)])
            return carry
        lax.fori_loop(0, NPT // CH, chunk, None)

    pl.when(c == 0)(lambda: half(src_hbm, ue_hbm, su_hbm, xsu_hbm))
    pl.when(c == 1)(lambda: half(dst_hbm, ie_hbm, si_hbm, xsi_hbm))


def _layer_body(xsu_hbm, xsi_hbm, src_hbm, dst_hbm, su_hbm, si_hbm,
                yu_hbm, yi_hbm,
                acc_sh, gidxs, sidxs, rowsv, svecv, semi, semg, sems):
    c = lax.axis_index("c")
    t = lax.axis_index("s")
    nbase = t * NPT
    rbase = t * RPT
    zbuf = rowsv.at[0].at[pl.ds(0, CH)]

    def zl(i, carry):
        for k in range(DIM // L):
            zbuf[i, pl.ds(k * L, L)] = jnp.zeros((L,), jnp.float32)
        return carry
    lax.fori_loop(0, CH, zl, None)

    def zc(i, carry):
        pltpu.sync_copy(zbuf, acc_sh.at[pl.ds(nbase + i * CH, CH)])
        return carry
    lax.fori_loop(0, NPT // CH, zc, None)
    plsc.subcore_barrier()

    def half(gat_hbm, sct_hbm, xsrc_hbm, s_hbm, y_hbm):
        def dchunk(cc, carry):
            rb = rbase + cc * CG
            di0 = pltpu.async_copy(gat_hbm.at[pl.ds(rb, CG)], gidxs, semi)
            di1 = pltpu.async_copy(sct_hbm.at[pl.ds(rb, CG)], sidxs, semi)
            di0.wait()
            di1.wait()
            gd = [None] * CG
            sd = [None] * CG
            gd[0] = pltpu.async_copy(xsrc_hbm.at[gidxs.at[0]],
                                     rowsv.at[0], semg)
            gd[1] = pltpu.async_copy(xsrc_hbm.at[gidxs.at[1]],
                                     rowsv.at[1], semg)
            for gg in range(CG):
                gd[gg].wait()
                sd[gg] = pltpu.async_copy(rowsv.at[gg % 3],
                                          acc_sh.at[sidxs.at[gg]],
                                          sems, add=True)
                if gg + 2 < CG:
                    if gg >= 1:
                        sd[gg - 1].wait()
                    gd[gg + 2] = pltpu.async_copy(
                        xsrc_hbm.at[gidxs.at[gg + 2]],
                        rowsv.at[(gg + 2) % 3], semg)
            sd[CG - 3].wait()
            sd[CG - 2].wait()
            sd[CG - 1].wait()
            return carry
        lax.fori_loop(0, NCH, dchunk, None)
        plsc.subcore_barrier()

        pltpu.sync_copy(s_hbm.at[pl.ds(nbase, NPT)], svecv)

        def chunk(cix, carry):
            r0 = nbase + cix * CH
            pltpu.sync_copy(acc_sh.at[pl.ds(r0, CH)], zbuf)

            def rgroup(i16, carry2):
                s16 = svecv[pl.ds(cix * CH + i16 * L, L)]
                for r in range(L):
                    row = i16 * L + r
                    s2 = s16[r] * s16[r]
                    for k in range(DIM // L):
                        sl = pl.ds(k * L, L)
                        zbuf[row, sl] = zbuf[row, sl] * s2
                return carry2
            lax.fori_loop(0, CH // L, rgroup, None)
            pltpu.sync_copy(zbuf, y_hbm.at[pl.ds(r0, CH)])
            return carry
        lax.fori_loop(0, NPT // CH, chunk, None)

    pl.when(c == 0)(lambda: half(dst_hbm, src_hbm, xsi_hbm, su_hbm, yu_hbm))
    pl.when(c == 1)(lambda: half(src_hbm, dst_hbm, xsu_hbm, si_hbm, yi_hbm))


def _score_body(users_hbm, items_hbm,
                xu0, xu1, xu2, xu3, xi0, xi1, xi2, xi3,
                su_hbm, si_hbm, out_hbm,
                uidx, iidx, uacc, iacc, gbuf, subuf, sibuf, dvec, sem):
    c = lax.axis_index("c")
    t = lax.axis_index("s")
    w = t * 2 + c
    base = w * PB

    pltpu.sync_copy(users_hbm.at[pl.ds(base, PB)], uidx)
    pltpu.sync_copy(items_hbm.at[pl.ds(base, PB)], iidx)

    def gather_sum(idxv, acc, tabs):
        pltpu.async_copy(tabs[0].at[idxv], acc, sem).wait()
        for tab in tabs[1:]:
            pltpu.async_copy(tab.at[idxv], gbuf, sem).wait()

            def al(r, carry):
                for k in range(DIM // L):
                    sl = pl.ds(k * L, L)
                    acc[r, sl] = acc[r, sl] + gbuf[r, sl]
                return carry
            lax.fori_loop(0, PB, al, None)

    gather_sum(uidx, uacc, (xu0, xu1, xu2, xu3))
    gather_sum(iidx, iacc, (xi0, xi1, xi2, xi3))

    lane = lax.iota(jnp.int32, L)

    def dgroup(g, carry):
        res = jnp.zeros((L,), jnp.float32)
        for r in range(L):
            row = g * L + r
            p = uacc[row, pl.ds(0, L)] * iacc[row, pl.ds(0, L)]
            for k in range(1, DIM // L):
                sl = pl.ds(k * L, L)
                p = p + uacc[row, sl] * iacc[row, sl]
            res = jnp.where(lane == r, jnp.sum(p), res)
        dvec[pl.ds(g * L, L)] = res
        return carry
    lax.fori_loop(0, PB // L, dgroup, None)

    pltpu.async_copy(su_hbm.at[uidx], subuf, sem).wait()
    pltpu.async_copy(si_hbm.at[iidx], sibuf, sem).wait()

    def fl(i, carry):
        sl = pl.ds(i * L, L)
        dvec[sl] = dvec[sl] / (jnp.float32(16.0) * subuf[sl] * sibuf[sl])
        return carry
    lax.fori_loop(0, PB // L, fl, None)
    pltpu.sync_copy(dvec, out_hbm.at[pl.ds(base, PB)])


def kernel(users, items, edge_index, user_emb, item_emb):
    src = edge_index[0].astype(jnp.int32)
    dst = edge_index[1].astype(jnp.int32)
    npad = EP - E
    padidx = NU + (jnp.arange(npad, dtype=jnp.int32) % (NP - NU))
    src_p = jnp.concatenate([src, padidx]).reshape(ER, 128)
    dst_p = jnp.concatenate([dst, padidx]).reshape(ER, 128)
    ue = jnp.zeros((NP, DIM), jnp.float32).at[:NU].set(user_emb)
    ie = jnp.zeros((NP, DIM), jnp.float32).at[:NI].set(item_emb)

    f32 = jnp.float32
    mesh = plsc.VectorSubcoreMesh(core_axis_name="c", subcore_axis_name="s")
    cparams = pltpu.CompilerParams(use_tc_tiling_on_sc=False,
                                   needs_layout_passes=False)
    rows2d = jax.ShapeDtypeStruct((NP, DIM), f32)
    vec1d = jax.ShapeDtypeStruct((NP,), f32)

    setup = pl.kernel(
        _setup_body,
        out_type=[vec1d, vec1d, rows2d, rows2d],
        mesh=mesh,
        compiler_params=cparams,
        scratch_types=[
            pltpu.VMEM_SHARED((NP,), f32),
            pltpu.VMEM((CG, 128), jnp.int32),
            pltpu.VMEM((128,), f32),
            pltpu.VMEM((NPT,), f32),
            pltpu.VMEM((NPT,), f32),
            pltpu.VMEM((CH, DIM), f32),
            pltpu.SemaphoreType.DMA,
            pltpu.SemaphoreType.DMA,
        ],
    )
    su, si, xsu, xsi = setup(src_p, dst_p, ue, ie)

    layer = pl.kernel(
        _layer_body,
        out_type=[rows2d, rows2d],
        mesh=mesh,
        compiler_params=cparams,
        scratch_types=[
            pltpu.VMEM_SHARED((NP, DIM), f32),
            pltpu.VMEM((CG, 128), jnp.int32),
            pltpu.VMEM((CG, 128), jnp.int32),
            pltpu.VMEM((3, 128, DIM), f32),
            pltpu.VMEM((NPT,), f32),
            pltpu.SemaphoreType.DMA,
            pltpu.SemaphoreType.DMA,
            pltpu.SemaphoreType.DMA,
        ],
    )
    xs_us = [xsu]
    xs_is = [xsi]
    for _ in range(3):
        xsu, xsi = layer(xsu, xsi, src_p, dst_p, su, si)
        xs_us.append(xsu)
        xs_is.append(xsi)

    score = pl.kernel(
        _score_body,
        out_type=jax.ShapeDtypeStruct((BATCH,), f32),
        mesh=mesh,
        compiler_params=cparams,
        scratch_types=[
            pltpu.VMEM((PB,), jnp.int32),
            pltpu.VMEM((PB,), jnp.int32),
            pltpu.VMEM((PB, DIM), f32),
            pltpu.VMEM((PB, DIM), f32),
            pltpu.VMEM((PB, DIM), f32),
            pltpu.VMEM((PB,), f32),
            pltpu.VMEM((PB,), f32),
            pltpu.VMEM((PB,), f32),
            pltpu.SemaphoreType.DMA,
        ],
    )
    return score(users.astype(jnp.int32), items.astype(jnp.int32),
                 *xs_us, *xs_is, su, si)


# speedup vs baseline: 1.0108x; 1.0108x over previous
import jax
import jax.numpy as jnp
from jax import lax
from jax.experimental import pallas as pl
from jax.experimental.pallas import tpu as pltpu
from jax.experimental.pallas import tpu_sc as plsc

NU = 25000
NI = 25000
NP = 25088
DIM = 64
E = 800000
ET = 50176
EP = ET * 16
ER = EP // 128
RPT = ER // 16
NB = 2
NG = RPT // NB
CG = 14
NCH = RPT // CG
NPT = NP // 16
CH = 112
BATCH = 4096
PB = BATCH // 32
L = 16


def _rsqrt16(x):
    i = lax.bitcast_convert_type(x, jnp.int32)
    i = jnp.int32(0x5F3759DF) - lax.shift_right_arithmetic(i, 1)
    y = lax.bitcast_convert_type(i, jnp.float32)
    for _ in range(3):
        y = y * (jnp.float32(1.5) - jnp.float32(0.5) * x * y * y)
    return y


def _setup_body(src_hbm, dst_hbm, ue_hbm, ie_hbm,
                su_hbm, si_hbm, xsu_hbm, xsi_hbm,
                deg_sh, idxc, onesv, degv, svecv, rowsv, semi, sems):
    c = lax.axis_index("c")
    t = lax.axis_index("s")
    nbase = t * NPT
    rbase = t * RPT

    for k in range(128 // L):
        onesv[pl.ds(k * L, L)] = jnp.ones((L,), jnp.float32)

    def zloop(i, carry):
        svecv[pl.ds(i * L, L)] = jnp.zeros((L,), jnp.float32)
        return carry
    lax.fori_loop(0, NPT // L, zloop, None)
    pltpu.sync_copy(svecv, deg_sh.at[pl.ds(nbase, NPT)])
    plsc.subcore_barrier()

    def half(idx_hbm, emb_hbm, s_hbm, xs_hbm):
        def gloop(g, carry):
            rb = rbase + g * CG
            pltpu.async_copy(idx_hbm.at[pl.ds(rb, CG)], idxc, semi).wait()
            for j in range(CG):
                pltpu.async_copy(onesv, deg_sh.at[idxc.at[j]], sems,
                                 add=True).wait()
            return carry
        lax.fori_loop(0, RPT // CG, gloop, None)
        plsc.subcore_barrier()

        pltpu.sync_copy(deg_sh.at[pl.ds(nbase, NPT)], degv)

        def sloop(i, carry):
            d = jnp.maximum(degv[pl.ds(i * L, L)], jnp.float32(1.0))
            svecv[pl.ds(i * L, L)] = _rsqrt16(d)
            return carry
        lax.fori_loop(0, NPT // L, sloop, None)
        pltpu.sync_copy(svecv, s_hbm.at[pl.ds(nbase, NPT)])

        nchk = NPT // CH
        pb = [rowsv.at[k] for k in range(3)]
        ind = [None] * nchk
        outd = [None] * nchk
        ind[0] = pltpu.async_copy(emb_hbm.at[pl.ds(nbase, CH)], pb[0], semi)
        for i in range(nchk):
            b = pb[i % 3]
            if i + 1 < nchk:
                if i >= 2:
                    outd[i - 2].wait()
                ind[i + 1] = pltpu.async_copy(
                    emb_hbm.at[pl.ds(nbase + (i + 1) * CH, CH)],
                    pb[(i + 1) % 3], semi)
            ind[i].wait()

            def rgroup(i16, carry2, i=i, b=b):
                s16 = svecv[pl.ds(i * CH + i16 * L, L)]
                for r in range(L):
                    row = i16 * L + r
                    sv = s16[r]
                    for k in range(DIM // L):
                        sl = pl.ds(k * L, L)
                        b[row, sl] = b[row, sl] * sv
                return carry2
            lax.fori_loop(0, CH // L, rgroup, None)
            outd[i] = pltpu.async_copy(
                b, xs_hbm.at[pl.ds(nbase + i * CH, CH)], sems)
        outd[nchk - 3].wait()
        outd[nchk - 2].wait()
        outd[nchk - 1].wait()

    pl.when(c == 0)(lambda: half(src_hbm, ue_hbm, su_hbm, xsu_hbm))
    pl.when(c == 1)(lambda: half(dst_hbm, ie_hbm, si_hbm, xsi_hbm))


def _layer_body(xsu_hbm, xsi_hbm, src_hbm, dst_hbm, su_hbm, si_hbm,
                yu_hbm, yi_hbm,
                acc_sh, gidxs, sidxs, rowsv, svecv, semi, semg, sems):
    c = lax.axis_index("c")
    t = lax.axis_index("s")
    nbase = t * NPT
    rbase = t * RPT
    zbuf = rowsv.at[0].at[pl.ds(0, CH)]

    def zl(i, carry):
        for k in range(DIM // L):
            zbuf[i, pl.ds(k * L, L)] = jnp.zeros((L,), jnp.float32)
        return carry
    lax.fori_loop(0, CH, zl, None)

    zd = [pltpu.async_copy(zbuf, acc_sh.at[pl.ds(nbase + i * CH, CH)], semi)
          for i in range(NPT // CH)]
    for d in zd:
        d.wait()
    plsc.subcore_barrier()

    def half(gat_hbm, sct_hbm, xsrc_hbm, s_hbm, y_hbm):
        def dchunk(cc, carry):
            rb = rbase + cc * CG
            di0 = pltpu.async_copy(gat_hbm.at[pl.ds(rb, CG)], gidxs, semi)
            di1 = pltpu.async_copy(sct_hbm.at[pl.ds(rb, CG)], sidxs, semi)
            di0.wait()
            di1.wait()
            gd = [None] * CG
            sd = [None] * CG
            gd[0] = pltpu.async_copy(xsrc_hbm.at[gidxs.at[0]],
                                     rowsv.at[0], semg)
            gd[1] = pltpu.async_copy(xsrc_hbm.at[gidxs.at[1]],
                                     rowsv.at[1], semg)
            for gg in range(CG):
                gd[gg].wait()
                sd[gg] = pltpu.async_copy(rowsv.at[gg % 3],
                                          acc_sh.at[sidxs.at[gg]],
                                          sems, add=True)
                if gg + 2 < CG:
                    if gg >= 1:
                        sd[gg - 1].wait()
                    gd[gg + 2] = pltpu.async_copy(
                        xsrc_hbm.at[gidxs.at[gg + 2]],
                        rowsv.at[(gg + 2) % 3], semg)
            sd[CG - 3].wait()
            sd[CG - 2].wait()
            sd[CG - 1].wait()
            return carry
        lax.fori_loop(0, NCH, dchunk, None)
        plsc.subcore_barrier()

        pltpu.sync_copy(s_hbm.at[pl.ds(nbase, NPT)], svecv)
        nchk = NPT // CH
        pb = [rowsv.at[k].at[pl.ds(0, CH)] for k in range(3)]
        ind = [None] * nchk
        outd = [None] * nchk
        ind[0] = pltpu.async_copy(acc_sh.at[pl.ds(nbase, CH)], pb[0], semg)
        for i in range(nchk):
            b = pb[i % 3]
            if i + 1 < nchk:
                if i >= 2:
                    outd[i - 2].wait()
                ind[i + 1] = pltpu.async_copy(
                    acc_sh.at[pl.ds(nbase + (i + 1) * CH, CH)],
                    pb[(i + 1) % 3], semg)
            ind[i].wait()

            def rgroup(i16, carry2, i=i, b=b):
                s16 = svecv[pl.ds(i * CH + i16 * L, L)]
                for r in range(L):
                    row = i16 * L + r
                    s2 = s16[r] * s16[r]
                    for k in range(DIM // L):
                        sl = pl.ds(k * L, L)
                        b[row, sl] = b[row, sl] * s2
                return carry2
            lax.fori_loop(0, CH // L, rgroup, None)
            outd[i] = pltpu.async_copy(
                b, y_hbm.at[pl.ds(nbase + i * CH, CH)], sems)
        outd[nchk - 3].wait()
        outd[nchk - 2].wait()
        outd[nchk - 1].wait()

    pl.when(c == 0)(lambda: half(dst_hbm, src_hbm, xsi_hbm, su_hbm, yu_hbm))
    pl.when(c == 1)(lambda: half(src_hbm, dst_hbm, xsu_hbm, si_hbm, yi_hbm))


def _score_body(users_hbm, items_hbm,
                xu0, xu1, xu2, xu3, xi0, xi1, xi2, xi3,
                su_hbm, si_hbm, out_hbm,
                uidx, iidx, uacc, iacc, gbuf, subuf, sibuf, dvec, sem):
    c = lax.axis_index("c")
    t = lax.axis_index("s")
    w = t * 2 + c
    base = w * PB

    pltpu.sync_copy(users_hbm.at[pl.ds(base, PB)], uidx)
    pltpu.sync_copy(items_hbm.at[pl.ds(base, PB)], iidx)

    def gather_sum(idxv, acc, tabs):
        pltpu.async_copy(tabs[0].at[idxv], acc, sem).wait()
        for tab in tabs[1:]:
            pltpu.async_copy(tab.at[idxv], gbuf, sem).wait()

            def al(r, carry):
                for k in range(DIM // L):
                    sl = pl.ds(k * L, L)
                    acc[r, sl] = acc[r, sl] + gbuf[r, sl]
                return carry
            lax.fori_loop(0, PB, al, None)

    gather_sum(uidx, uacc, (xu0, xu1, xu2, xu3))
    gather_sum(iidx, iacc, (xi0, xi1, xi2, xi3))

    lane = lax.iota(jnp.int32, L)

    def dgroup(g, carry):
        res = jnp.zeros((L,), jnp.float32)
        for r in range(L):
            row = g * L + r
            p = uacc[row, pl.ds(0, L)] * iacc[row, pl.ds(0, L)]
            for k in range(1, DIM // L):
                sl = pl.ds(k * L, L)
                p = p + uacc[row, sl] * iacc[row, sl]
            res = jnp.where(lane == r, jnp.sum(p), res)
        dvec[pl.ds(g * L, L)] = res
        return carry
    lax.fori_loop(0, PB // L, dgroup, None)

    pltpu.async_copy(su_hbm.at[uidx], subuf, sem).wait()
    pltpu.async_copy(si_hbm.at[iidx], sibuf, sem).wait()

    def fl(i, carry):
        sl = pl.ds(i * L, L)
        dvec[sl] = dvec[sl] / (jnp.float32(16.0) * subuf[sl] * sibuf[sl])
        return carry
    lax.fori_loop(0, PB // L, fl, None)
    pltpu.sync_copy(dvec, out_hbm.at[pl.ds(base, PB)])


def kernel(users, items, edge_index, user_emb, item_emb):
    src = edge_index[0].astype(jnp.int32)
    dst = edge_index[1].astype(jnp.int32)
    npad = EP - E
    padidx = NU + (jnp.arange(npad, dtype=jnp.int32) % (NP - NU))
    src_p = jnp.concatenate([src, padidx]).reshape(ER, 128)
    dst_p = jnp.concatenate([dst, padidx]).reshape(ER, 128)
    ue = jnp.zeros((NP, DIM), jnp.float32).at[:NU].set(user_emb)
    ie = jnp.zeros((NP, DIM), jnp.float32).at[:NI].set(item_emb)

    f32 = jnp.float32
    mesh = plsc.VectorSubcoreMesh(core_axis_name="c", subcore_axis_name="s")
    cparams = pltpu.CompilerParams(use_tc_tiling_on_sc=False,
                                   needs_layout_passes=False)
    rows2d = jax.ShapeDtypeStruct((NP, DIM), f32)
    vec1d = jax.ShapeDtypeStruct((NP,), f32)

    setup = pl.kernel(
        _setup_body,
        out_type=[vec1d, vec1d, rows2d, rows2d],
        mesh=mesh,
        compiler_params=cparams,
        scratch_types=[
            pltpu.VMEM_SHARED((NP,), f32),
            pltpu.VMEM((CG, 128), jnp.int32),
            pltpu.VMEM((128,), f32),
            pltpu.VMEM((NPT,), f32),
            pltpu.VMEM((NPT,), f32),
            pltpu.VMEM((3, CH, DIM), f32),
            pltpu.SemaphoreType.DMA,
            pltpu.SemaphoreType.DMA,
        ],
    )
    su, si, xsu, xsi = setup(src_p, dst_p, ue, ie)

    layer = pl.kernel(
        _layer_body,
        out_type=[rows2d, rows2d],
        mesh=mesh,
        compiler_params=cparams,
        scratch_types=[
            pltpu.VMEM_SHARED((NP, DIM), f32),
            pltpu.VMEM((CG, 128), jnp.int32),
            pltpu.VMEM((CG, 128), jnp.int32),
            pltpu.VMEM((3, 128, DIM), f32),
            pltpu.VMEM((NPT,), f32),
            pltpu.SemaphoreType.DMA,
            pltpu.SemaphoreType.DMA,
            pltpu.SemaphoreType.DMA,
        ],
    )
    xs_us = [xsu]
    xs_is = [xsi]
    for _ in range(3):
        xsu, xsi = layer(xsu, xsi, src_p, dst_p, su, si)
        xs_us.append(xsu)
        xs_is.append(xsi)

    score = pl.kernel(
        _score_body,
        out_type=jax.ShapeDtypeStruct((BATCH,), f32),
        mesh=mesh,
        compiler_params=cparams,
        scratch_types=[
            pltpu.VMEM((PB,), jnp.int32),
            pltpu.VMEM((PB,), jnp.int32),
            pltpu.VMEM((PB, DIM), f32),
            pltpu.VMEM((PB, DIM), f32),
            pltpu.VMEM((PB, DIM), f32),
            pltpu.VMEM((PB,), f32),
            pltpu.VMEM((PB,), f32),
            pltpu.VMEM((PB,), f32),
            pltpu.SemaphoreType.DMA,
        ],
    )
    return score(users.astype(jnp.int32), items.astype(jnp.int32),
                 *xs_us, *xs_is, su, si)


# speedup vs baseline: 1.0504x; 1.0391x over previous
import jax
import jax.numpy as jnp
from jax import lax
from jax.experimental import pallas as pl
from jax.experimental.pallas import tpu as pltpu
from jax.experimental.pallas import tpu_sc as plsc

NU = 25000
NI = 25000
NP = 25088
DIM = 64
E = 800000
ET = 50176
EP = ET * 16
ER = EP // 128
RPT = ER // 16
CG = 14
NCH = RPT // CG
NPT = NP // 16
CH = 112
BATCH = 4096
PB = BATCH // 32
L = 16


def _rsqrt16(x):
    i = lax.bitcast_convert_type(x, jnp.int32)
    i = jnp.int32(0x5F3759DF) - lax.shift_right_arithmetic(i, 1)
    y = lax.bitcast_convert_type(i, jnp.float32)
    for _ in range(3):
        y = y * (jnp.float32(1.5) - jnp.float32(0.5) * x * y * y)
    return y


def _main_body(eidx_hbm, emb_hbm, x_hbm, s_hbm,
               acc_sh, deg_sh, gidxs, sidxs, rowsv, svc, onesv,
               semi, semg, sems):
    c = lax.axis_index("c")
    t = lax.axis_index("s")
    nbase = t * NPT
    rbase = t * RPT
    myside = lax.rem(c + 1, 2)
    nchk = NPT // CH
    pb = [rowsv.at[k].at[pl.ds(0, CH)] for k in range(3)]
    zbuf = pb[0]

    for k in range(128 // L):
        onesv[pl.ds(k * L, L)] = jnp.ones((L,), jnp.float32)

    for k in range(CH // L):
        svc[0, pl.ds(k * L, L)] = jnp.zeros((L,), jnp.float32)
    zd = [pltpu.async_copy(svc.at[0],
                           deg_sh.at[pl.ds(nbase + i * CH, CH)], semi)
          for i in range(nchk)]
    for d in zd:
        d.wait()
    plsc.subcore_barrier()

    def dloop(g, carry):
        rb = rbase + g * CG
        pltpu.async_copy(eidx_hbm.at[myside, pl.ds(rb, CG)], gidxs,
                         semi).wait()
        for j in range(CG):
            pltpu.async_copy(onesv, deg_sh.at[gidxs.at[j]], sems,
                             add=True).wait()
        return carry
    lax.fori_loop(0, NCH, dloop, None)
    plsc.subcore_barrier()

    ind = [None] * nchk
    outd = [None] * nchk
    ind[0] = pltpu.async_copy(emb_hbm.at[myside, pl.ds(nbase, CH)],
                              pb[0], semg)
    for i in range(nchk):
        b = pb[i % 3]
        if i + 1 < nchk:
            if i >= 2:
                outd[i - 2].wait()
            ind[i + 1] = pltpu.async_copy(
                emb_hbm.at[myside, pl.ds(nbase + (i + 1) * CH, CH)],
                pb[(i + 1) % 3], semg)
        r0 = nbase + i * CH
        pltpu.sync_copy(deg_sh.at[pl.ds(r0, CH)], svc.at[0])

        def sgrp(k, carry, ):
            d = jnp.maximum(svc[0, pl.ds(k * L, L)], jnp.float32(1.0))
            svc[1, pl.ds(k * L, L)] = _rsqrt16(d)
            return carry
        lax.fori_loop(0, CH // L, sgrp, None)
        pltpu.sync_copy(svc.at[1], s_hbm.at[myside, pl.ds(r0, CH)])
        ind[i].wait()

        def rgroup(i16, carry, b=b):
            s16 = svc[1, pl.ds(i16 * L, L)]
            for r in range(L):
                row = i16 * L + r
                sv = s16[r]
                for k in range(DIM // L):
                    sl = pl.ds(k * L, L)
                    b[row, sl] = b[row, sl] * sv
            return carry
        lax.fori_loop(0, CH // L, rgroup, None)
        outd[i] = pltpu.async_copy(
            b, x_hbm.at[0, myside, pl.ds(r0, CH)], sems)
    outd[nchk - 3].wait()
    outd[nchk - 2].wait()
    outd[nchk - 1].wait()
    plsc.subcore_barrier()

    def stage(l, carry):
        sig = lax.rem(l - 1 + c, 2)
        gam = 1 - sig

        def zl(i, carry2):
            for k in range(DIM // L):
                zbuf[i, pl.ds(k * L, L)] = jnp.zeros((L,), jnp.float32)
            return carry2
        lax.fori_loop(0, CH, zl, None)
        zd2 = [pltpu.async_copy(zbuf,
                                acc_sh.at[pl.ds(nbase + i * CH, CH)], semi)
               for i in range(nchk)]
        for d in zd2:
            d.wait()
        plsc.subcore_barrier()

        def dchunk(cc, carry2):
            rb = rbase + cc * CG
            di0 = pltpu.async_copy(eidx_hbm.at[gam, pl.ds(rb, CG)],
                                   gidxs, semi)
            di1 = pltpu.async_copy(eidx_hbm.at[sig, pl.ds(rb, CG)],
                                   sidxs, semi)
            di0.wait()
            di1.wait()
            gd = [None] * CG
            sd = [None] * CG
            gsrc = x_hbm.at[l - 1, gam]
            gd[0] = pltpu.async_copy(gsrc.at[gidxs.at[0]],
                                     rowsv.at[0], semg)
            gd[1] = pltpu.async_copy(gsrc.at[gidxs.at[1]],
                                     rowsv.at[1], semg)
            for gg in range(CG):
                gd[gg].wait()
                sd[gg] = pltpu.async_copy(rowsv.at[gg % 3],
                                          acc_sh.at[sidxs.at[gg]],
                                          sems, add=True)
                if gg + 2 < CG:
                    if gg >= 1:
                        sd[gg - 1].wait()
                    gd[gg + 2] = pltpu.async_copy(
                        gsrc.at[gidxs.at[gg + 2]],
                        rowsv.at[(gg + 2) % 3], semg)
            sd[CG - 3].wait()
            sd[CG - 2].wait()
            sd[CG - 1].wait()
            return carry2
        lax.fori_loop(0, NCH, dchunk, None)
        plsc.subcore_barrier()

        ind2 = [None] * nchk
        outd2 = [None] * nchk
        ind2[0] = pltpu.async_copy(acc_sh.at[pl.ds(nbase, CH)],
                                   pb[0], semg)
        for i in range(nchk):
            b = pb[i % 3]
            if i + 1 < nchk:
                if i >= 2:
                    outd2[i - 2].wait()
                ind2[i + 1] = pltpu.async_copy(
                    acc_sh.at[pl.ds(nbase + (i + 1) * CH, CH)],
                    pb[(i + 1) % 3], semg)
            r0 = nbase + i * CH
            pltpu.sync_copy(s_hbm.at[sig, pl.ds(r0, CH)], svc.at[0])

            def sqg(k, carry2):
                v = svc[0, pl.ds(k * L, L)]
                svc[1, pl.ds(k * L, L)] = v * v
                return carry2
            lax.fori_loop(0, CH // L, sqg, None)
            ind2[i].wait()

            def rgroup2(i16, carry2, b=b):
                s16 = svc[1, pl.ds(i16 * L, L)]
                for r in range(L):
                    row = i16 * L + r
                    sv = s16[r]
                    for k in range(DIM // L):
                        sl = pl.ds(k * L, L)
                        b[row, sl] = b[row, sl] * sv
                return carry2
            lax.fori_loop(0, CH // L, rgroup2, None)
            outd2[i] = pltpu.async_copy(
                b, x_hbm.at[l, sig, pl.ds(r0, CH)], sems)
        outd2[nchk - 3].wait()
        outd2[nchk - 2].wait()
        outd2[nchk - 1].wait()
        plsc.subcore_barrier()
        return carry
    lax.fori_loop(1, 4, stage, None)


def _score_body(users_hbm, items_hbm, x_hbm, s_hbm, out_hbm,
                uidx, iidx, uacc, iacc, gbuf, subuf, sibuf, dvec, sem):
    c = lax.axis_index("c")
    t = lax.axis_index("s")
    w = t * 2 + c
    base = w * PB

    pltpu.sync_copy(users_hbm.at[pl.ds(base, PB)], uidx)
    pltpu.sync_copy(items_hbm.at[pl.ds(base, PB)], iidx)

    def gather_sum(idxv, acc, side):
        pltpu.async_copy(x_hbm.at[0, side].at[idxv], acc, sem).wait()
        for ll in range(1, 4):
            pltpu.async_copy(x_hbm.at[ll, side].at[idxv], gbuf, sem).wait()

            def al(r, carry):
                for k in range(DIM // L):
                    sl = pl.ds(k * L, L)
                    acc[r, sl] = acc[r, sl] + gbuf[r, sl]
                return carry
            lax.fori_loop(0, PB, al, None)

    gather_sum(uidx, uacc, 0)
    gather_sum(iidx, iacc, 1)

    lane = lax.iota(jnp.int32, L)

    def dgroup(g, carry):
        res = jnp.zeros((L,), jnp.float32)
        for r in range(L):
            row = g * L + r
            p = uacc[row, pl.ds(0, L)] * iacc[row, pl.ds(0, L)]
            for k in range(1, DIM // L):
                sl = pl.ds(k * L, L)
                p = p + uacc[row, sl] * iacc[row, sl]
            res = jnp.where(lane == r, jnp.sum(p), res)
        dvec[pl.ds(g * L, L)] = res
        return carry
    lax.fori_loop(0, PB // L, dgroup, None)

    pltpu.async_copy(s_hbm.at[0].at[uidx], subuf, sem).wait()
    pltpu.async_copy(s_hbm.at[1].at[iidx], sibuf, sem).wait()

    def fl(i, carry):
        sl = pl.ds(i * L, L)
        dvec[sl] = dvec[sl] / (jnp.float32(16.0) * subuf[sl] * sibuf[sl])
        return carry
    lax.fori_loop(0, PB // L, fl, None)
    pltpu.sync_copy(dvec, out_hbm.at[pl.ds(base, PB)])


def kernel(users, items, edge_index, user_emb, item_emb):
    src = edge_index[0].astype(jnp.int32)
    dst = edge_index[1].astype(jnp.int32)
    npad = EP - E
    padidx = NU + (jnp.arange(npad, dtype=jnp.int32) % (NP - NU))
    src_p = jnp.concatenate([src, padidx]).reshape(ER, 128)
    dst_p = jnp.concatenate([dst, padidx]).reshape(ER, 128)
    eidx = jnp.stack([src_p, dst_p])
    ue = jnp.zeros((NP, DIM), jnp.float32).at[:NU].set(user_emb)
    ie = jnp.zeros((NP, DIM), jnp.float32).at[:NI].set(item_emb)
    emb = jnp.stack([ue, ie])

    f32 = jnp.float32
    mesh = plsc.VectorSubcoreMesh(core_axis_name="c", subcore_axis_name="s")
    cparams = pltpu.CompilerParams(use_tc_tiling_on_sc=False,
                                   needs_layout_passes=False)

    main = pl.kernel(
        _main_body,
        out_type=[jax.ShapeDtypeStruct((4, 2, NP, DIM), f32),
                  jax.ShapeDtypeStruct((2, NP), f32)],
        mesh=mesh,
        compiler_params=cparams,
        scratch_types=[
            pltpu.VMEM_SHARED((NP, DIM), f32),
            pltpu.VMEM_SHARED((NP,), f32),
            pltpu.VMEM((CG, 128), jnp.int32),
            pltpu.VMEM((CG, 128), jnp.int32),
            pltpu.VMEM((3, 128, DIM), f32),
            pltpu.VMEM((2, CH), f32),
            pltpu.VMEM((128,), f32),
            pltpu.SemaphoreType.DMA,
            pltpu.SemaphoreType.DMA,
            pltpu.SemaphoreType.DMA,
        ],
    )
    x_all, s_all = main(eidx, emb)

    score = pl.kernel(
        _score_body,
        out_type=jax.ShapeDtypeStruct((BATCH,), f32),
        mesh=mesh,
        compiler_params=cparams,
        scratch_types=[
            pltpu.VMEM((PB,), jnp.int32),
            pltpu.VMEM((PB,), jnp.int32),
            pltpu.VMEM((PB, DIM), f32),
            pltpu.VMEM((PB, DIM), f32),
            pltpu.VMEM((PB, DIM), f32),
            pltpu.VMEM((PB,), f32),
            pltpu.VMEM((PB,), f32),
            pltpu.VMEM((PB,), f32),
            pltpu.SemaphoreType.DMA,
        ],
    )
    return score(users.astype(jnp.int32), items.astype(jnp.int32),
                 x_all, s_all)


# speedup vs baseline: 1.0604x; 1.0095x over previous
import jax
import jax.numpy as jnp
from jax import lax
from jax.experimental import pallas as pl
from jax.experimental.pallas import tpu as pltpu
from jax.experimental.pallas import tpu_sc as plsc

NU = 25000
NI = 25000
NP = 25088
DIM = 64
E = 800000
ET = 50176
EP = ET * 16
ER = EP // 128
RPT = ER // 16
CG = 14
NCH = RPT // CG
NPT = NP // 16
CH = 112
BATCH = 4096
PB = BATCH // 32
L = 16


def _rsqrt16(x):
    i = lax.bitcast_convert_type(x, jnp.int32)
    i = jnp.int32(0x5F3759DF) - lax.shift_right_arithmetic(i, 1)
    y = lax.bitcast_convert_type(i, jnp.float32)
    for _ in range(3):
        y = y * (jnp.float32(1.5) - jnp.float32(0.5) * x * y * y)
    return y


def _main_body(eidx_hbm, emb_hbm, x_hbm, s_hbm,
               acc_sh, deg_sh, gidxs, sidxs, rowsv, svc, onesv,
               semi, semg, sems):
    c = lax.axis_index("c")
    t = lax.axis_index("s")
    nbase = t * NPT
    rbase = t * RPT
    myside = lax.rem(c + 1, 2)
    nchk = NPT // CH
    pb = [rowsv.at[k].at[pl.ds(0, CH)] for k in range(3)]
    zbuf = pb[0]

    for k in range(128 // L):
        onesv[pl.ds(k * L, L)] = jnp.ones((L,), jnp.float32)

    for k in range(CH // L):
        svc[0, pl.ds(k * L, L)] = jnp.zeros((L,), jnp.float32)
    zd = [pltpu.async_copy(svc.at[0],
                           deg_sh.at[pl.ds(nbase + i * CH, CH)], semi)
          for i in range(nchk)]
    for d in zd:
        d.wait()
    plsc.subcore_barrier()

    def dloop(g, carry):
        rb = rbase + g * CG
        pltpu.async_copy(eidx_hbm.at[myside, pl.ds(rb, CG)], gidxs,
                         semi).wait()
        for j in range(CG):
            pltpu.async_copy(onesv, deg_sh.at[gidxs.at[j]], sems,
                             add=True).wait()
        return carry
    lax.fori_loop(0, NCH, dloop, None)
    plsc.subcore_barrier()

    ind = [None] * nchk
    outd = [None] * nchk
    ind[0] = pltpu.async_copy(emb_hbm.at[myside, pl.ds(nbase, CH)],
                              pb[0], semg)
    for i in range(nchk):
        b = pb[i % 3]
        if i + 1 < nchk:
            if i >= 2:
                outd[i - 2].wait()
            ind[i + 1] = pltpu.async_copy(
                emb_hbm.at[myside, pl.ds(nbase + (i + 1) * CH, CH)],
                pb[(i + 1) % 3], semg)
        r0 = nbase + i * CH
        pltpu.sync_copy(deg_sh.at[pl.ds(r0, CH)], svc.at[0])

        def sgrp(k, carry, ):
            d = jnp.maximum(svc[0, pl.ds(k * L, L)], jnp.float32(1.0))
            svc[1, pl.ds(k * L, L)] = _rsqrt16(d)
            return carry
        lax.fori_loop(0, CH // L, sgrp, None)
        pltpu.sync_copy(svc.at[1], s_hbm.at[myside, pl.ds(r0, CH)])
        ind[i].wait()

        def rgroup(i16, carry, b=b):
            s16 = svc[1, pl.ds(i16 * L, L)]
            for r in range(L):
                row = i16 * L + r
                sv = s16[r]
                for k in range(DIM // L):
                    sl = pl.ds(k * L, L)
                    b[row, sl] = b[row, sl] * sv
            return carry
        lax.fori_loop(0, CH // L, rgroup, None)
        outd[i] = pltpu.async_copy(
            b, x_hbm.at[0, myside, pl.ds(r0, CH)], sems)
    outd[nchk - 3].wait()
    outd[nchk - 2].wait()
    outd[nchk - 1].wait()
    plsc.subcore_barrier()

    def stage(l, carry):
        sig = lax.rem(l - 1 + c, 2)
        gam = 1 - sig

        def zl(i, carry2):
            for k in range(DIM // L):
                zbuf[i, pl.ds(k * L, L)] = jnp.zeros((L,), jnp.float32)
            return carry2
        lax.fori_loop(0, CH, zl, None)
        zd2 = [pltpu.async_copy(zbuf,
                                acc_sh.at[pl.ds(nbase + i * CH, CH)], semi)
               for i in range(nchk)]
        for d in zd2:
            d.wait()
        plsc.subcore_barrier()

        def dchunk(cc, carry2):
            rb = rbase + cc * CG
            di0 = pltpu.async_copy(eidx_hbm.at[gam, pl.ds(rb, CG)],
                                   gidxs, semi)
            di1 = pltpu.async_copy(eidx_hbm.at[sig, pl.ds(rb, CG)],
                                   sidxs, semi)
            di0.wait()
            di1.wait()
            gd = [None] * CG
            sd = [None] * CG
            gsrc = x_hbm.at[l - 1, gam]
            gd[0] = pltpu.async_copy(gsrc.at[gidxs.at[0]],
                                     rowsv.at[0], semg)
            gd[1] = pltpu.async_copy(gsrc.at[gidxs.at[1]],
                                     rowsv.at[1], semg)
            for gg in range(CG):
                gd[gg].wait()
                sd[gg] = pltpu.async_copy(rowsv.at[gg % 3],
                                          acc_sh.at[sidxs.at[gg]],
                                          sems, add=True)
                if gg + 2 < CG:
                    if gg >= 1:
                        sd[gg - 1].wait()
                    gd[gg + 2] = pltpu.async_copy(
                        gsrc.at[gidxs.at[gg + 2]],
                        rowsv.at[(gg + 2) % 3], semg)
            sd[CG - 3].wait()
            sd[CG - 2].wait()
            sd[CG - 1].wait()
            return carry2
        lax.fori_loop(0, NCH, dchunk, None)
        plsc.subcore_barrier()

        ind2 = [None] * nchk
        outd2 = [None] * nchk
        ind2[0] = pltpu.async_copy(acc_sh.at[pl.ds(nbase, CH)],
                                   pb[0], semg)
        for i in range(nchk):
            b = pb[i % 3]
            if i + 1 < nchk:
                if i >= 2:
                    outd2[i - 2].wait()
                ind2[i + 1] = pltpu.async_copy(
                    acc_sh.at[pl.ds(nbase + (i + 1) * CH, CH)],
                    pb[(i + 1) % 3], semg)
            r0 = nbase + i * CH
            pltpu.sync_copy(s_hbm.at[sig, pl.ds(r0, CH)], svc.at[0])

            def sqg(k, carry2):
                v = svc[0, pl.ds(k * L, L)]
                svc[1, pl.ds(k * L, L)] = v * v
                return carry2
            lax.fori_loop(0, CH // L, sqg, None)
            ind2[i].wait()

            def rgroup2(i16, carry2, b=b):
                s16 = svc[1, pl.ds(i16 * L, L)]
                for r in range(L):
                    row = i16 * L + r
                    sv = s16[r]
                    for k in range(DIM // L):
                        sl = pl.ds(k * L, L)
                        b[row, sl] = b[row, sl] * sv
                return carry2
            lax.fori_loop(0, CH // L, rgroup2, None)
            outd2[i] = pltpu.async_copy(
                b, x_hbm.at[l, sig, pl.ds(r0, CH)], sems)
        outd2[nchk - 3].wait()
        outd2[nchk - 2].wait()
        outd2[nchk - 1].wait()
        plsc.subcore_barrier()
        return carry
    lax.fori_loop(1, 4, stage, None)


def _score_body(users_hbm, items_hbm, x_hbm, s_hbm, out_hbm,
                uidx, iidx, uacc, iacc, gbuf, subuf, sibuf, dvec, sem):
    c = lax.axis_index("c")
    t = lax.axis_index("s")
    w = t * 2 + c
    base = w * PB

    pltpu.sync_copy(users_hbm.at[pl.ds(base, PB)], uidx)
    pltpu.sync_copy(items_hbm.at[pl.ds(base, PB)], iidx)

    def gather_sum(idxv, acc, side):
        pltpu.async_copy(x_hbm.at[0, side].at[idxv], acc, sem).wait()
        for ll in range(1, 4):
            pltpu.async_copy(x_hbm.at[ll, side].at[idxv], gbuf, sem).wait()

            def al(r, carry):
                for k in range(DIM // L):
                    sl = pl.ds(k * L, L)
                    acc[r, sl] = acc[r, sl] + gbuf[r, sl]
                return carry
            lax.fori_loop(0, PB, al, None)

    gather_sum(uidx, uacc, 0)
    gather_sum(iidx, iacc, 1)

    lane = lax.iota(jnp.int32, L)

    def dgroup(g, carry):
        res = jnp.zeros((L,), jnp.float32)
        for r in range(L):
            row = g * L + r
            p = uacc[row, pl.ds(0, L)] * iacc[row, pl.ds(0, L)]
            for k in range(1, DIM // L):
                sl = pl.ds(k * L, L)
                p = p + uacc[row, sl] * iacc[row, sl]
            res = jnp.where(lane == r, jnp.sum(p), res)
        dvec[pl.ds(g * L, L)] = res
        return carry
    lax.fori_loop(0, PB // L, dgroup, None)

    pltpu.async_copy(s_hbm.at[0].at[uidx], subuf, sem).wait()
    pltpu.async_copy(s_hbm.at[1].at[iidx], sibuf, sem).wait()

    def fl(i, carry):
        sl = pl.ds(i * L, L)
        dvec[sl] = dvec[sl] / (jnp.float32(16.0) * subuf[sl] * sibuf[sl])
        return carry
    lax.fori_loop(0, PB // L, fl, None)
    pltpu.sync_copy(dvec, out_hbm.at[pl.ds(base, PB)])


def kernel(users, items, edge_index, user_emb, item_emb):
    npad = EP - E
    padidx = NU + (jnp.arange(npad, dtype=jnp.int32) % (NP - NU))
    eidx = jnp.concatenate(
        [edge_index.astype(jnp.int32),
         jnp.broadcast_to(padidx, (2, npad))], axis=1).reshape(2, ER, 128)
    emb = (jnp.zeros((2, NP, DIM), jnp.float32)
           .at[0, :NU].set(user_emb).at[1, :NI].set(item_emb))

    f32 = jnp.float32
    mesh = plsc.VectorSubcoreMesh(core_axis_name="c", subcore_axis_name="s")
    cparams = pltpu.CompilerParams(use_tc_tiling_on_sc=False,
                                   needs_layout_passes=False)

    main = pl.kernel(
        _main_body,
        out_type=[jax.ShapeDtypeStruct((4, 2, NP, DIM), f32),
                  jax.ShapeDtypeStruct((2, NP), f32)],
        mesh=mesh,
        compiler_params=cparams,
        scratch_types=[
            pltpu.VMEM_SHARED((NP, DIM), f32),
            pltpu.VMEM_SHARED((NP,), f32),
            pltpu.VMEM((CG, 128), jnp.int32),
            pltpu.VMEM((CG, 128), jnp.int32),
            pltpu.VMEM((3, 128, DIM), f32),
            pltpu.VMEM((2, CH), f32),
            pltpu.VMEM((128,), f32),
            pltpu.SemaphoreType.DMA,
            pltpu.SemaphoreType.DMA,
            pltpu.SemaphoreType.DMA,
        ],
    )
    x_all, s_all = main(eidx, emb)

    score = pl.kernel(
        _score_body,
        out_type=jax.ShapeDtypeStruct((BATCH,), f32),
        mesh=mesh,
        compiler_params=cparams,
        scratch_types=[
            pltpu.VMEM((PB,), jnp.int32),
            pltpu.VMEM((PB,), jnp.int32),
            pltpu.VMEM((PB, DIM), f32),
            pltpu.VMEM((PB, DIM), f32),
            pltpu.VMEM((PB, DIM), f32),
            pltpu.VMEM((PB,), f32),
            pltpu.VMEM((PB,), f32),
            pltpu.VMEM((PB,), f32),
            pltpu.SemaphoreType.DMA,
        ],
    )
    return score(users.astype(jnp.int32), items.astype(jnp.int32),
                 x_all, s_all)
